# baseline (device time: 49259 ns/iter reference)
import jax
import jax.numpy as jnp
from jax import lax
from jax.experimental import pallas as pl
from jax.experimental.pallas import tpu as pltpu

N_DEV = 4
CAPACITY = 102


def kernel(x, router_W, route_idx, expert_W):
    del router_W
    n_tok, d_model = x.shape
    e_per, _, d_hidden = expert_W.shape
    n_exp = N_DEV * e_per

    def body(x_ref, route_ref, w_ref, out_ref,
             wg_ref, cg_ref, wsend, wrecv, csend, crecv):
        me = lax.axis_index("i")
        left = lax.rem(me + N_DEV - 1, N_DEV)
        right = lax.rem(me + 1, N_DEV)

        barrier_sem = pltpu.get_barrier_semaphore()
        for nbr in [left, right]:
            pl.semaphore_signal(
                barrier_sem, inc=1,
                device_id=(nbr,), device_id_type=pl.DeviceIdType.MESH,
            )
        pl.semaphore_wait(barrier_sem, 2)

        rm = (route_ref[...] ==
              lax.broadcasted_iota(jnp.int32, (1, n_exp), 1)
              ).astype(jnp.float32)
        cg_ref[pl.ds(me, 1)] = jnp.sum(rm, axis=0, keepdims=True)

        wg_ref[pl.ds(me * e_per, e_per)] = w_ref[...].astype(jnp.bfloat16)

        for h in range(N_DEV - 1):
            slot = lax.rem(me - h + N_DEV, N_DEV)
            w_rdma = pltpu.make_async_remote_copy(
                src_ref=wg_ref.at[pl.ds(slot * e_per, e_per)],
                dst_ref=wg_ref.at[pl.ds(slot * e_per, e_per)],
                send_sem=wsend.at[h], recv_sem=wrecv.at[h],
                device_id=(right,), device_id_type=pl.DeviceIdType.MESH,
            )
            c_rdma = pltpu.make_async_remote_copy(
                src_ref=cg_ref.at[pl.ds(slot, 1)],
                dst_ref=cg_ref.at[pl.ds(slot, 1)],
                send_sem=csend.at[h], recv_sem=crecv.at[h],
                device_id=(right,), device_id_type=pl.DeviceIdType.MESH,
            )
            w_rdma.start()
            c_rdma.start()
            w_rdma.wait()
            c_rdma.wait()

        cg = cg_ref[...]
        sidx = lax.broadcasted_iota(jnp.int32, (N_DEV, 1), 0)
        base = jnp.sum(jnp.where(sidx < me, cg, 0.0), axis=0,
                       keepdims=True)
        ii = lax.broadcasted_iota(jnp.int32, (n_tok, n_tok), 0)
        jj = lax.broadcasted_iota(jnp.int32, (n_tok, n_tok), 1)
        ltri = (jj < ii).astype(jnp.bfloat16)
        prefix = lax.dot_general(
            ltri, rm.astype(jnp.bfloat16),
            (((1,), (0,)), ((), ())),
            preferred_element_type=jnp.float32,
        )
        rank = base + prefix
        mask = jnp.where((rm > 0.0) & (rank < float(CAPACITY)),
                         1.0, 0.0).astype(jnp.bfloat16)

        xb = x_ref[...].astype(jnp.bfloat16)
        acc = jnp.zeros((n_tok, d_hidden), jnp.float32)
        for e in range(n_exp):
            xe = xb * mask[:, e:e + 1]
            acc = acc + lax.dot_general(
                xe, wg_ref[e],
                (((1,), (0,)), ((), ())),
                preferred_element_type=jnp.float32,
            )
        out_ref[...] = acc

    return pl.pallas_call(
        body,
        out_shape=jax.ShapeDtypeStruct((n_tok, d_hidden), jnp.float32),
        in_specs=[
            pl.BlockSpec(memory_space=pltpu.VMEM),
            pl.BlockSpec(memory_space=pltpu.VMEM),
            pl.BlockSpec(memory_space=pltpu.VMEM),
        ],
        out_specs=pl.BlockSpec(memory_space=pltpu.VMEM),
        scratch_shapes=[
            pltpu.VMEM((n_exp, d_model, d_hidden), jnp.bfloat16),
            pltpu.VMEM((N_DEV, n_exp), jnp.float32),
            pltpu.SemaphoreType.DMA((N_DEV - 1,)),
            pltpu.SemaphoreType.DMA((N_DEV - 1,)),
            pltpu.SemaphoreType.DMA((N_DEV - 1,)),
            pltpu.SemaphoreType.DMA((N_DEV - 1,)),
        ],
        compiler_params=pltpu.CompilerParams(collective_id=0),
    )(x, route_idx, expert_W)


# device time: 29321 ns/iter; 1.6800x vs baseline; 1.6800x over previous
import jax
import jax.numpy as jnp
from jax import lax
from jax.experimental import pallas as pl
from jax.experimental.pallas import tpu as pltpu

N_DEV = 4
CAPACITY = 102


def kernel(x, router_W, route_idx, expert_W):
    del router_W
    n_tok, d_model = x.shape
    e_per, _, d_hidden = expert_W.shape
    n_exp = N_DEV * e_per
    e_half = e_per // 2

    def body(x_ref, route_ref, w_ref, out_ref,
             wg_ref, cg_ref, wsend, wrecv, csend, crecv):
        me = lax.axis_index("i")
        left = lax.rem(me + N_DEV - 1, N_DEV)
        right = lax.rem(me + 1, N_DEV)
        across = lax.rem(me + 2, N_DEV)

        barrier_sem = pltpu.get_barrier_semaphore()
        for nbr in [left, right, across]:
            pl.semaphore_signal(
                barrier_sem, inc=1,
                device_id=(nbr,), device_id_type=pl.DeviceIdType.MESH,
            )
        pl.semaphore_wait(barrier_sem, 3)

        rm = (route_ref[...] ==
              lax.broadcasted_iota(jnp.int32, (1, n_exp), 1)
              ).astype(jnp.float32)
        cg_ref[pl.ds(me, 1)] = jnp.sum(rm, axis=0, keepdims=True)

        wg_ref[pl.ds(me * e_per, e_per)] = w_ref[...].astype(jnp.bfloat16)

        def copy(src_slice, dst_slice, sems, k, target):
            return pltpu.make_async_remote_copy(
                src_ref=wg_ref.at[src_slice] if sems is wsend else
                        cg_ref.at[src_slice],
                dst_ref=wg_ref.at[dst_slice] if sems is wsend else
                        cg_ref.at[dst_slice],
                send_sem=(wsend if sems is wsend else csend).at[k],
                recv_sem=(wrecv if sems is wsend else crecv).at[k],
                device_id=(target,), device_id_type=pl.DeviceIdType.MESH,
            )

        my_c = pl.ds(me, 1)
        c_r = copy(my_c, my_c, csend, 0, right)
        c_l = copy(my_c, my_c, csend, 1, left)
        c_a = copy(my_c, my_c, csend, 2, across)
        c_r.start()
        c_l.start()
        c_a.start()

        my_b = pl.ds(me * e_per, e_per)
        w_r1 = copy(my_b, my_b, wsend, 0, right)
        w_l1 = copy(my_b, my_b, wsend, 1, left)
        w_r1.start()
        w_l1.start()

        c_r.wait_recv()
        c_l.wait_recv()
        c_a.wait_recv()

        cg = cg_ref[...]
        sidx = lax.broadcasted_iota(jnp.int32, (N_DEV, 1), 0)
        base = jnp.sum(jnp.where(sidx < me, cg, 0.0), axis=0,
                       keepdims=True)
        ii = lax.broadcasted_iota(jnp.int32, (n_tok, n_tok), 0)
        jj = lax.broadcasted_iota(jnp.int32, (n_tok, n_tok), 1)
        ltri = (jj < ii).astype(jnp.bfloat16)
        prefix = lax.dot_general(
            ltri, rm.astype(jnp.bfloat16),
            (((1,), (0,)), ((), ())),
            preferred_element_type=jnp.float32,
        )
        rank = base + prefix
        mask = jnp.where((rm > 0.0) & (rank < float(CAPACITY)),
                         1.0, 0.0).astype(jnp.bfloat16)

        xb = x_ref[...].astype(jnp.bfloat16)

        def block_contrib(acc, slot_off):
            blk = lax.rem(me + slot_off + N_DEV, N_DEV)
            for j in range(e_per):
                e_idx = blk * e_per + j
                col = jnp.sum(
                    mask * (lax.broadcasted_iota(jnp.int32, (1, n_exp), 1)
                            == e_idx).astype(jnp.bfloat16),
                    axis=1, keepdims=True)
                w_e = wg_ref[pl.ds(e_idx, 1)][0]
                acc = acc + lax.dot_general(
                    xb * col, w_e,
                    (((1,), (0,)), ((), ())),
                    preferred_element_type=jnp.float32,
                )
            return acc

        acc = jnp.zeros((n_tok, d_hidden), jnp.float32)
        acc = block_contrib(acc, 0)

        lb = lax.rem(me - 1 + N_DEV, N_DEV)
        rb = lax.rem(me + 1, N_DEV)
        w_r1.wait_recv()
        lo = pl.ds(lb * e_per, e_half)
        w_r2 = copy(lo, lo, wsend, 2, right)
        w_r2.start()
        w_l1.wait_recv()
        hi = pl.ds(rb * e_per + e_half, e_half)
        w_l2 = copy(hi, hi, wsend, 3, left)
        w_l2.start()

        acc = block_contrib(acc, -1)
        acc = block_contrib(acc, 1)

        w_r2.wait_recv()
        w_l2.wait_recv()
        acc = block_contrib(acc, 2)
        out_ref[...] = acc

        for d in (c_r, c_l, c_a, w_r1, w_l1, w_r2, w_l2):
            d.wait_send()

    return pl.pallas_call(
        body,
        out_shape=jax.ShapeDtypeStruct((n_tok, d_hidden), jnp.float32),
        in_specs=[
            pl.BlockSpec(memory_space=pltpu.VMEM),
            pl.BlockSpec(memory_space=pltpu.VMEM),
            pl.BlockSpec(memory_space=pltpu.VMEM),
        ],
        out_specs=pl.BlockSpec(memory_space=pltpu.VMEM),
        scratch_shapes=[
            pltpu.VMEM((n_exp, d_model, d_hidden), jnp.bfloat16),
            pltpu.VMEM((N_DEV, n_exp), jnp.float32),
            pltpu.SemaphoreType.DMA((4,)),
            pltpu.SemaphoreType.DMA((4,)),
            pltpu.SemaphoreType.DMA((3,)),
            pltpu.SemaphoreType.DMA((3,)),
        ],
        compiler_params=pltpu.CompilerParams(collective_id=0),
    )(x, route_idx, expert_W)


# device time: 27478 ns/iter; 1.7927x vs baseline; 1.0671x over previous
import jax
import jax.numpy as jnp
from jax import lax
from jax.experimental import pallas as pl
from jax.experimental.pallas import tpu as pltpu

N_DEV = 4
CAPACITY = 102


def kernel(x, router_W, route_idx, expert_W):
    del router_W
    n_tok, d_model = x.shape
    e_per, _, d_hidden = expert_W.shape
    n_exp = N_DEV * e_per
    e_half = e_per // 2

    def body(x_ref, route_ref, w_ref, out_ref,
             wg_ref, cg_ref, wsend, wrecv, csend, crecv):
        me = lax.axis_index("i")
        left = lax.rem(me + N_DEV - 1, N_DEV)
        right = lax.rem(me + 1, N_DEV)
        across = lax.rem(me + 2, N_DEV)

        barrier_sem = pltpu.get_barrier_semaphore()
        for nbr in [left, right, across]:
            pl.semaphore_signal(
                barrier_sem, inc=1,
                device_id=(nbr,), device_id_type=pl.DeviceIdType.MESH,
            )
        pl.semaphore_wait(barrier_sem, 3)

        def wcopy(slice_, k, target):
            return pltpu.make_async_remote_copy(
                src_ref=wg_ref.at[slice_], dst_ref=wg_ref.at[slice_],
                send_sem=wsend.at[k], recv_sem=wrecv.at[k],
                device_id=(target,), device_id_type=pl.DeviceIdType.MESH,
            )

        def ccopy(k, target):
            return pltpu.make_async_remote_copy(
                src_ref=cg_ref.at[pl.ds(me, 1)],
                dst_ref=cg_ref.at[pl.ds(me, 1)],
                send_sem=csend.at[k], recv_sem=crecv.at[k],
                device_id=(target,), device_id_type=pl.DeviceIdType.MESH,
            )

        rm = (route_ref[...] ==
              lax.broadcasted_iota(jnp.int32, (1, n_exp), 1)
              ).astype(jnp.float32)
        cg_ref[pl.ds(me, 1)] = jnp.sum(rm, axis=0, keepdims=True)
        c_r = ccopy(0, right)
        c_l = ccopy(1, left)
        c_a = ccopy(2, across)
        c_r.start()
        c_l.start()
        c_a.start()

        wg_ref[pl.ds(me * e_per, e_per)] = w_ref[...].astype(jnp.bfloat16)
        my_lo = pl.ds(me * e_per, e_half)
        my_hi = pl.ds(me * e_per + e_half, e_half)
        w_r1a = wcopy(my_lo, 0, right)
        w_r1b = wcopy(my_hi, 1, right)
        w_l1a = wcopy(my_hi, 2, left)
        w_l1b = wcopy(my_lo, 3, left)
        w_r1a.start()
        w_r1b.start()
        w_l1a.start()
        w_l1b.start()

        c_r.wait_recv()
        c_l.wait_recv()
        c_a.wait_recv()

        cg = cg_ref[...]
        sidx = lax.broadcasted_iota(jnp.int32, (N_DEV, 1), 0)
        base = jnp.sum(jnp.where(sidx < me, cg, 0.0), axis=0,
                       keepdims=True)
        ii = lax.broadcasted_iota(jnp.int32, (n_tok, n_tok), 0)
        jj = lax.broadcasted_iota(jnp.int32, (n_tok, n_tok), 1)
        ltri = (jj < ii).astype(jnp.bfloat16)
        prefix = lax.dot_general(
            ltri, rm.astype(jnp.bfloat16),
            (((1,), (0,)), ((), ())),
            preferred_element_type=jnp.float32,
        )
        rank = base + prefix
        mask = jnp.where((rm > 0.0) & (rank < float(CAPACITY)),
                         1.0, 0.0).astype(jnp.bfloat16)

        xb = x_ref[...].astype(jnp.bfloat16)

        def half_contrib(acc, blk, half):
            for j in range(e_half):
                e_idx = blk * e_per + half * e_half + j
                col = jnp.sum(
                    mask * (lax.broadcasted_iota(jnp.int32, (1, n_exp), 1)
                            == e_idx).astype(jnp.bfloat16),
                    axis=1, keepdims=True)
                w_e = wg_ref[pl.ds(e_idx, 1)][0]
                acc = acc + lax.dot_general(
                    xb * col, w_e,
                    (((1,), (0,)), ((), ())),
                    preferred_element_type=jnp.float32,
                )
            return acc

        def block_contrib(acc, slot_off):
            blk = lax.rem(me + slot_off + N_DEV, N_DEV)
            acc = half_contrib(acc, blk, 0)
            return half_contrib(acc, blk, 1)

        acc = jnp.zeros((n_tok, d_hidden), jnp.float32)
        acc = block_contrib(acc, 0)

        lb = lax.rem(me - 1 + N_DEV, N_DEV)
        rb = lax.rem(me + 1, N_DEV)
        w_r1a.wait_recv()
        lo = pl.ds(lb * e_per, e_half)
        w_r2 = wcopy(lo, 4, right)
        w_r2.start()
        w_l1a.wait_recv()
        hi = pl.ds(rb * e_per + e_half, e_half)
        w_l2 = wcopy(hi, 5, left)
        w_l2.start()

        w_r1b.wait_recv()
        acc = block_contrib(acc, -1)
        w_l1b.wait_recv()
        acc = block_contrib(acc, 1)

        ab = lax.rem(me + 2, N_DEV)
        w_r2.wait_recv()
        acc = half_contrib(acc, ab, 0)
        w_l2.wait_recv()
        acc = half_contrib(acc, ab, 1)
        out_ref[...] = acc

        for d in (c_r, c_l, c_a, w_r1a, w_r1b, w_l1a, w_l1b, w_r2, w_l2):
            d.wait_send()

    return pl.pallas_call(
        body,
        out_shape=jax.ShapeDtypeStruct((n_tok, d_hidden), jnp.float32),
        in_specs=[
            pl.BlockSpec(memory_space=pltpu.VMEM),
            pl.BlockSpec(memory_space=pltpu.VMEM),
            pl.BlockSpec(memory_space=pltpu.VMEM),
        ],
        out_specs=pl.BlockSpec(memory_space=pltpu.VMEM),
        scratch_shapes=[
            pltpu.VMEM((n_exp, d_model, d_hidden), jnp.bfloat16),
            pltpu.VMEM((N_DEV, n_exp), jnp.float32),
            pltpu.SemaphoreType.DMA((6,)),
            pltpu.SemaphoreType.DMA((6,)),
            pltpu.SemaphoreType.DMA((3,)),
            pltpu.SemaphoreType.DMA((3,)),
        ],
        compiler_params=pltpu.CompilerParams(collective_id=0),
    )(x, route_idx, expert_W)
